# baseline (speedup 1.0000x reference)
import functools

import jax
import jax.numpy as jnp
from jax import lax
from jax.experimental import pallas as pl
from jax.experimental.pallas import tpu as pltpu
from jax.experimental.pallas import tpu_sc as plsc

_NUM_LABELS = 3
_D = 1024
_B = 16384
_NC = 2
_NS = 16
_NW = _NC * _NS
_BPW0 = 448
_BPW1 = 576
_DRAIN = 32


def _sc_embedding_lookup(ids, table):
    mesh = plsc.VectorSubcoreMesh(core_axis_name="c", subcore_axis_name="s")

    @functools.partial(
        pl.kernel,
        mesh=mesh,
        out_type=jax.ShapeDtypeStruct((_B, _D), jnp.float32),
        scratch_types=[
            pltpu.VMEM((_BPW1,), jnp.int32),
            pltpu.VMEM((_NUM_LABELS, _D), jnp.float32),
            pltpu.VMEM((_DRAIN, _D), jnp.float32),
            pltpu.SemaphoreType.DMA,
        ],
    )
    def k(ids_hbm, table_hbm, out_hbm, idx_v, table_v, dummy_v, sem):
        c = lax.axis_index("c")
        s = lax.axis_index("s")
        bpw = jnp.where(c == 0, _BPW0, _BPW1)
        base = jnp.where(c == 0, s * _BPW0, _NS * _BPW0 + s * _BPW1)
        pltpu.sync_copy(ids_hbm.at[pl.ds(base, _BPW1)], idx_v)
        pltpu.sync_copy(table_hbm, table_v)

        def issue_group(g, carry):
            ids16 = idx_v[pl.ds(g * 16, 16)]
            for j in range(16):
                rid = ids16[j]
                pltpu.async_copy(
                    table_v.at[pl.ds(rid, 1)],
                    out_hbm.at[pl.ds(base + g * 16 + j, 1)],
                    sem,
                )
            return carry

        lax.fori_loop(0, bpw // 16, issue_group, 0)

        def drain(r, carry):
            pltpu.make_async_copy(out_hbm.at[pl.ds(0, _DRAIN)], dummy_v, sem).wait()
            return carry

        lax.fori_loop(0, bpw // _DRAIN, drain, 0)

    return k(ids, table)


def kernel(sentiment_ids, embedding_table):
    ids = sentiment_ids.astype(jnp.int32)
    return _sc_embedding_lookup(ids, embedding_table.astype(jnp.float32))
